# baseline (speedup 1.0000x reference)
import jax
import jax.numpy as jnp
from jax.experimental import pallas as pl

TH = 0.5
ALPHA = 1.0
BETA = 0.5


def _histc(x, bins, min_val, max_val):
    x = x.reshape(-1)
    width = (max_val - min_val) / bins
    idx = jnp.clip(jnp.floor((x - min_val) / width), 0, bins - 1).astype(jnp.int32)
    valid = (x >= min_val) & (x <= max_val)
    return jnp.zeros((bins,), dtype=jnp.float32).at[jnp.where(valid, idx, 0)].add(jnp.where(valid, 1.0, 0.0))


def _otsu(tensor):
    flat = tensor.reshape(tensor.shape[0], tensor.shape[1], -1)
    hist = jax.vmap(jax.vmap(lambda v: _histc(v, 256, 0.0, 255.0)))(flat)
    prob = hist / jnp.sum(hist, axis=2, keepdims=True)
    cum_prob = jnp.cumsum(prob, axis=2)
    cum_mean = jnp.cumsum(prob * jnp.arange(256, dtype=jnp.float32)[None, None, :], axis=2)
    global_mean = cum_mean[:, :, -1:]
    numerator = (global_mean * cum_prob - cum_mean) ** 2
    denominator = cum_prob * (1.0 - cum_prob)
    between_class_variance = numerator / denominator
    threshold = jnp.argmax(between_class_variance, axis=2)
    return tensor > threshold[:, :, None, None].astype(tensor.dtype)


def _calc_hist(image, min_val, max_val):
    h_r = _histc(image[:, 0], 256, min_val, max_val)
    h_g = _histc(image[:, 1], 256, min_val, max_val)
    h_b = _histc(image[:, 2], 256, min_val, max_val)
    histogram = jnp.stack((h_r, h_g, h_b), axis=0)
    return histogram / jnp.sum(histogram)


def _ce(image1, image2, min_val, max_val):
    eps = 1e-10
    p1 = jnp.clip(_calc_hist(image1, min_val, max_val), eps, None)
    p2 = jnp.clip(_calc_hist(image2, min_val, max_val), eps, None)
    return -jnp.sum(p1 * jnp.log(p2))


def _identity_kernel(x_ref, o_ref):
    o_ref[...] = x_ref[...]


def kernel(cam_cln, cam_haz, img_haz):
    b, c, h, w = img_haz.shape
    mask_cln = jnp.repeat(_otsu(cam_cln * 255.0), 3, axis=1)
    mask_haz = jnp.repeat(_otsu(cam_haz * 255.0), 3, axis=1)
    img_fore_cln = jnp.where(mask_cln, img_haz, 0.0)
    img_back_cln = jnp.where(mask_cln, 0.0, img_haz)
    img_fore_haz = jnp.where(mask_haz, img_haz, 0.0)
    img_back_haz = jnp.where(mask_haz, 0.0, img_haz)
    mn = jnp.min(img_haz)
    mx = jnp.max(img_haz)
    mn = jnp.where(mn == 0, mn + 0.001, mn)
    ce_positive = _ce(img_fore_cln, img_fore_haz, mn, mx) + _ce(img_back_cln, img_back_haz, mn, mx)
    ce_negative = -(_ce(img_fore_cln, img_back_haz, mn, mx) + _ce(img_back_cln, img_fore_haz, mn, mx))
    loss = ALPHA * ce_positive + BETA * ce_negative
    out = pl.pallas_call(
        _identity_kernel,
        out_shape=jax.ShapeDtypeStruct((8, 128), jnp.float32),
    )(jnp.broadcast_to(loss, (8, 128)))
    return out[0, 0]


# speedup vs baseline: 51.3196x; 51.3196x over previous
import jax
import jax.numpy as jnp
from jax import lax
from jax.experimental import pallas as pl
from jax.experimental.pallas import tpu as pltpu
from jax.experimental.pallas import tpu_sc as plsc

NW = 32
L = 16
CAM_PIX = 147456
CAM_HALF = CAM_PIX // 2
IMG_WORDS = 3538944
IMG_PER_TILE = IMG_WORDS // NW
K1_CH = 9216
K2_QTR = CAM_PIX // 4
K2_CH = 4608
CAM_W = 0.99609375


def _k1_body(cams_hbm, img_hbm, hist_out, mm_out, dbuf, hscr, fbuf, mmbuf):
    wid = lax.axis_index("s") * 2 + lax.axis_index("c")
    lane = lax.iota(jnp.int32, L) * 256
    zeros16 = jnp.zeros((L,), jnp.float32)
    ones16 = jnp.ones((L,), jnp.float32)

    def zero_hist(i, _):
        hscr[pl.ds(i * L, L)] = zeros16
        return 0
    lax.fori_loop(0, 256, zero_hist, 0)

    cam_off = (wid // 2) * CAM_PIX + (wid % 2) * CAM_HALF

    def cam_chunk(k, _):
        pltpu.sync_copy(cams_hbm.at[pl.ds(cam_off + k * K1_CH, K1_CH)], dbuf)

        def inner(i, _):
            x = dbuf[pl.ds(i * L, L)]
            v = x * 255.0
            q = v / CAM_W
            qi = jnp.clip(q.astype(jnp.int32), 0, 255)
            plsc.addupdate_scatter(hscr, [lane + qi], ones16)
            return 0
        lax.fori_loop(0, K1_CH // L, inner, 0)
        return 0
    lax.fori_loop(0, CAM_HALF // K1_CH, cam_chunk, 0)

    img_off = wid * IMG_PER_TILE

    def mm_chunk(k, carry):
        pltpu.sync_copy(img_hbm.at[pl.ds(img_off + k * K1_CH, K1_CH)], dbuf)

        def inner(i, c):
            a, b = c
            x = dbuf[pl.ds(i * L, L)]
            return (jnp.minimum(a, x), jnp.maximum(b, x))
        return lax.fori_loop(0, K1_CH // L, inner, carry)

    mn_acc, mx_acc = lax.fori_loop(
        0, IMG_PER_TILE // K1_CH, mm_chunk,
        (jnp.full((L,), jnp.inf, jnp.float32),
         jnp.full((L,), -jnp.inf, jnp.float32)))
    mmbuf[pl.ds(0, L)] = mn_acc
    mmbuf[pl.ds(L, L)] = mx_acc
    pltpu.sync_copy(mmbuf, mm_out.at[wid])

    def red(g, _):
        acc = zeros16
        for l in range(L):
            acc = acc + hscr[pl.ds(l * 256 + g * L, L)]
        fbuf[pl.ds(g * L, L)] = acc
        return 0
    lax.fori_loop(0, 16, red, 0)
    pltpu.sync_copy(fbuf, hist_out.at[wid])


def _k2_body(cams_hbm, img_hbm, params_hbm, hist_out,
             pbuf, cc, ch, v0, v1, v2, hscr, fbuf):
    wid = lax.axis_index("s") * 2 + lax.axis_index("c")
    b = wid // 4
    part = wid % 4
    lane = lax.iota(jnp.int32, L) * 3072
    zeros16 = jnp.zeros((L,), jnp.float32)
    ones16 = jnp.ones((L,), jnp.float32)

    pltpu.sync_copy(params_hbm.at[wid], pbuf)
    t_c = pbuf[pl.ds(0, L)]
    t_h = pbuf[pl.ds(16, L)]
    mnv = pbuf[pl.ds(32, L)]
    mxv = pbuf[pl.ds(48, L)]
    wdv = pbuf[pl.ds(64, L)]

    def zero_hist(i, _):
        hscr[pl.ds(i * L, L)] = zeros16
        return 0
    lax.fori_loop(0, 3072, zero_hist, 0)

    cam_base = b * CAM_PIX + part * K2_QTR
    img_base0 = (b * 3 + 0) * CAM_PIX + part * K2_QTR
    img_base1 = (b * 3 + 1) * CAM_PIX + part * K2_QTR
    img_base2 = (b * 3 + 2) * CAM_PIX + part * K2_QTR

    def chunk(k, _):
        off = k * K2_CH
        pltpu.sync_copy(cams_hbm.at[pl.ds(cam_base + off, K2_CH)], cc)
        pltpu.sync_copy(cams_hbm.at[pl.ds(1179648 + cam_base + off, K2_CH)], ch)
        pltpu.sync_copy(img_hbm.at[pl.ds(img_base0 + off, K2_CH)], v0)
        pltpu.sync_copy(img_hbm.at[pl.ds(img_base1 + off, K2_CH)], v1)
        pltpu.sync_copy(img_hbm.at[pl.ds(img_base2 + off, K2_CH)], v2)

        def inner(i, _):
            sl = pl.ds(i * L, L)
            mc = (cc[sl] * 255.0) > t_c
            mh = (ch[sl] * 255.0) > t_h
            base = (jnp.where(mc, jnp.full((L,), 512, jnp.int32), jnp.zeros((L,), jnp.int32))
                    + jnp.where(mh, jnp.full((L,), 256, jnp.int32), jnp.zeros((L,), jnp.int32))
                    + lane)
            for ci, vb in ((0, v0), (1, v1), (2, v2)):
                x = vb[sl]
                valid = (x >= mnv) & (x <= mxv)
                q = (x - mnv) / wdv
                qi = jnp.clip(q.astype(jnp.int32), 0, 255)
                plsc.addupdate_scatter(hscr, [base + ci * 1024 + qi], ones16,
                                       mask=valid)
            return 0
        lax.fori_loop(0, K2_CH // L, inner, 0)
        return 0
    lax.fori_loop(0, K2_QTR // K2_CH, chunk, 0)

    def red(g, _):
        acc = zeros16
        for l in range(L):
            acc = acc + hscr[pl.ds(l * 3072 + g * L, L)]
        fbuf[pl.ds(g * L, L)] = acc
        return 0
    lax.fori_loop(0, 192, red, 0)
    pltpu.sync_copy(fbuf, hist_out.at[wid])


def _ce_body(fc_ref, bc_ref, fh_ref, bh_ref, o_ref):
    eps = 1e-10
    rows = lax.broadcasted_iota(jnp.int32, (8, 256), 0)
    valid = rows < 3

    def prep(h):
        s = jnp.sum(h)
        return jnp.clip(h / s, eps, None)

    pfc = prep(fc_ref[...])
    pbc = prep(bc_ref[...])
    pfh = prep(fh_ref[...])
    pbh = prep(bh_ref[...])
    lfh = jnp.log(pfh)
    lbh = jnp.log(pbh)

    def ce(pa, lb):
        return -jnp.sum(jnp.where(valid, pa * lb, 0.0))

    ce_pos = ce(pfc, lfh) + ce(pbc, lbh)
    ce_neg = -(ce(pfc, lbh) + ce(pbc, lfh))
    o_ref[...] = jnp.full((8, 128), 1.0 * ce_pos + 0.5 * ce_neg)


def _otsu_threshold(hist):
    prob = hist / jnp.sum(hist, axis=2, keepdims=True)
    cum_prob = jnp.cumsum(prob, axis=2)
    cum_mean = jnp.cumsum(prob * jnp.arange(256, dtype=jnp.float32)[None, None, :], axis=2)
    global_mean = cum_mean[:, :, -1:]
    numerator = (global_mean * cum_prob - cum_mean) ** 2
    denominator = cum_prob * (1.0 - cum_prob)
    between_class_variance = numerator / denominator
    return jnp.argmax(between_class_variance, axis=2)


def kernel(cam_cln, cam_haz, img_haz):
    mesh = plsc.VectorSubcoreMesh(core_axis_name="c", subcore_axis_name="s")
    cams_flat = jnp.concatenate([cam_cln.reshape(-1), cam_haz.reshape(-1)])
    img_flat = img_haz.reshape(-1)

    k1 = pl.kernel(
        _k1_body,
        out_type=[jax.ShapeDtypeStruct((NW, 256), jnp.float32),
                  jax.ShapeDtypeStruct((NW, 32), jnp.float32)],
        mesh=mesh,
        scratch_types=[pltpu.VMEM((K1_CH,), jnp.float32),
                       pltpu.VMEM((4096,), jnp.float32),
                       pltpu.VMEM((256,), jnp.float32),
                       pltpu.VMEM((32,), jnp.float32)],
        compiler_params=pltpu.CompilerParams(needs_layout_passes=False),
    )
    hist_part, mm = k1(cams_flat, img_flat)

    hist16 = hist_part.reshape(16, 2, 256).sum(axis=1)
    t_cln = _otsu_threshold(hist16[:8].reshape(8, 1, 256))
    t_haz = _otsu_threshold(hist16[8:].reshape(8, 1, 256))
    t_cln_f = t_cln.reshape(8).astype(jnp.float32)
    t_haz_f = t_haz.reshape(8).astype(jnp.float32)

    mn = jnp.min(mm[:, :16])
    mx = jnp.max(mm[:, 16:])
    mn = jnp.where(mn == 0, mn + 0.001, mn)
    width = (mx - mn) / 256

    row_b = jnp.arange(NW) // 4
    params = jnp.stack([
        t_cln_f[row_b], t_haz_f[row_b],
        jnp.broadcast_to(mn, (NW,)), jnp.broadcast_to(mx, (NW,)),
        jnp.broadcast_to(width, (NW,)),
    ], axis=1)
    params = jnp.broadcast_to(params[:, :, None], (NW, 5, 16)).reshape(NW, 80)

    k2 = pl.kernel(
        _k2_body,
        out_type=[jax.ShapeDtypeStruct((NW, 3072), jnp.float32)],
        mesh=mesh,
        scratch_types=[pltpu.VMEM((80,), jnp.float32),
                       pltpu.VMEM((K2_CH,), jnp.float32),
                       pltpu.VMEM((K2_CH,), jnp.float32),
                       pltpu.VMEM((K2_CH,), jnp.float32),
                       pltpu.VMEM((K2_CH,), jnp.float32),
                       pltpu.VMEM((K2_CH,), jnp.float32),
                       pltpu.VMEM((49152,), jnp.float32),
                       pltpu.VMEM((3072,), jnp.float32)],
        compiler_params=pltpu.CompilerParams(needs_layout_passes=False),
    )
    (hist2_part,) = k2(cams_flat, img_flat, params)

    H = hist2_part.sum(axis=0).reshape(3, 4, 256)
    fc = H[:, 2] + H[:, 3]
    bc = H[:, 0] + H[:, 1]
    fh = H[:, 1] + H[:, 3]
    bh = H[:, 0] + H[:, 2]

    def pad8(x):
        return jnp.concatenate([x, jnp.zeros((5, 256), jnp.float32)], axis=0)

    out = pl.pallas_call(
        _ce_body,
        out_shape=jax.ShapeDtypeStruct((8, 128), jnp.float32),
    )(pad8(fc), pad8(bc), pad8(fh), pad8(bh))
    return out[0, 0]


# speedup vs baseline: 54.5325x; 1.0626x over previous
import jax
import jax.numpy as jnp
from jax import lax
from jax.experimental import pallas as pl
from jax.experimental.pallas import tpu as pltpu
from jax.experimental.pallas import tpu_sc as plsc

NW = 32
L = 16
CAM_PIX = 147456
CAM_HALF = CAM_PIX // 2
IMG_WORDS = 3538944
IMG_PER_TILE = IMG_WORDS // NW
K1_CH = 9216
K2_QTR = CAM_PIX // 4
K2_CH = 4608
CAM_W = 0.99609375


def _k1_body(cams_hbm, img_hbm, hist_out, mm_out, dbuf, hscr, fbuf, mmbuf):
    wid = lax.axis_index("s") * 2 + lax.axis_index("c")
    lane = lax.iota(jnp.int32, L) * 256
    zeros16 = jnp.zeros((L,), jnp.float32)
    ones16 = jnp.ones((L,), jnp.float32)

    def zero_hist(i, _):
        hscr[pl.ds(i * L, L)] = zeros16
        return 0
    lax.fori_loop(0, 256, zero_hist, 0)

    cam_off = (wid // 2) * CAM_PIX + (wid % 2) * CAM_HALF

    U = 8

    def cam_chunk(k, _):
        pltpu.sync_copy(cams_hbm.at[pl.ds(cam_off + k * K1_CH, K1_CH)], dbuf)

        def inner(i, _):
            for u in range(U):
                x = dbuf[pl.ds((i * U + u) * L, L)]
                v = x * 255.0
                q = v / CAM_W
                qi = jnp.clip(q.astype(jnp.int32), 0, 255)
                plsc.addupdate_scatter(hscr, [lane + qi], ones16)
            return 0
        lax.fori_loop(0, K1_CH // (L * U), inner, 0)
        return 0
    lax.fori_loop(0, CAM_HALF // K1_CH, cam_chunk, 0)

    img_off = wid * IMG_PER_TILE

    def mm_chunk(k, carry):
        pltpu.sync_copy(img_hbm.at[pl.ds(img_off + k * K1_CH, K1_CH)], dbuf)

        def inner(i, c):
            a, b = c
            for u in range(U):
                x = dbuf[pl.ds((i * U + u) * L, L)]
                a = jnp.minimum(a, x)
                b = jnp.maximum(b, x)
            return (a, b)
        return lax.fori_loop(0, K1_CH // (L * U), inner, carry)

    mn_acc, mx_acc = lax.fori_loop(
        0, IMG_PER_TILE // K1_CH, mm_chunk,
        (jnp.full((L,), jnp.inf, jnp.float32),
         jnp.full((L,), -jnp.inf, jnp.float32)))
    mmbuf[pl.ds(0, L)] = mn_acc
    mmbuf[pl.ds(L, L)] = mx_acc
    pltpu.sync_copy(mmbuf, mm_out.at[wid])

    def red(g, _):
        acc = zeros16
        for l in range(L):
            acc = acc + hscr[pl.ds(l * 256 + g * L, L)]
        fbuf[pl.ds(g * L, L)] = acc
        return 0
    lax.fori_loop(0, 16, red, 0)
    pltpu.sync_copy(fbuf, hist_out.at[wid])


def _k2_body(cams_hbm, img_hbm, params_hbm, hist_out,
             pbuf, cc, ch, v0, v1, v2, hscr, fbuf):
    wid = lax.axis_index("s") * 2 + lax.axis_index("c")
    b = wid // 4
    part = wid % 4
    lane = lax.iota(jnp.int32, L) * 3072
    zeros16 = jnp.zeros((L,), jnp.float32)
    ones16 = jnp.ones((L,), jnp.float32)

    pltpu.sync_copy(params_hbm.at[wid], pbuf)
    t_c = pbuf[pl.ds(0, L)]
    t_h = pbuf[pl.ds(16, L)]
    mnv = pbuf[pl.ds(32, L)]
    mxv = pbuf[pl.ds(48, L)]
    wdv = pbuf[pl.ds(64, L)]

    def zero_hist(i, _):
        hscr[pl.ds(i * L, L)] = zeros16
        return 0
    lax.fori_loop(0, 3072, zero_hist, 0)

    cam_base = b * CAM_PIX + part * K2_QTR
    img_base0 = (b * 3 + 0) * CAM_PIX + part * K2_QTR
    img_base1 = (b * 3 + 1) * CAM_PIX + part * K2_QTR
    img_base2 = (b * 3 + 2) * CAM_PIX + part * K2_QTR

    def chunk(k, _):
        off = k * K2_CH
        pltpu.sync_copy(cams_hbm.at[pl.ds(cam_base + off, K2_CH)], cc)
        pltpu.sync_copy(cams_hbm.at[pl.ds(1179648 + cam_base + off, K2_CH)], ch)
        pltpu.sync_copy(img_hbm.at[pl.ds(img_base0 + off, K2_CH)], v0)
        pltpu.sync_copy(img_hbm.at[pl.ds(img_base1 + off, K2_CH)], v1)
        pltpu.sync_copy(img_hbm.at[pl.ds(img_base2 + off, K2_CH)], v2)

        def inner(i, _):
            for u in range(4):
                sl = pl.ds((i * 4 + u) * L, L)
                mc = (cc[sl] * 255.0) > t_c
                mh = (ch[sl] * 255.0) > t_h
                base = (jnp.where(mc, jnp.full((L,), 512, jnp.int32), jnp.zeros((L,), jnp.int32))
                        + jnp.where(mh, jnp.full((L,), 256, jnp.int32), jnp.zeros((L,), jnp.int32))
                        + lane)
                for ci, vb in ((0, v0), (1, v1), (2, v2)):
                    x = vb[sl]
                    valid = (x >= mnv) & (x <= mxv)
                    q = (x - mnv) / wdv
                    qi = jnp.clip(q.astype(jnp.int32), 0, 255)
                    plsc.addupdate_scatter(hscr, [base + ci * 1024 + qi], ones16,
                                           mask=valid)
            return 0
        lax.fori_loop(0, K2_CH // (L * 4), inner, 0)
        return 0
    lax.fori_loop(0, K2_QTR // K2_CH, chunk, 0)

    def red(g, _):
        acc = zeros16
        for l in range(L):
            acc = acc + hscr[pl.ds(l * 3072 + g * L, L)]
        fbuf[pl.ds(g * L, L)] = acc
        return 0
    lax.fori_loop(0, 192, red, 0)
    pltpu.sync_copy(fbuf, hist_out.at[wid])


def _ce_body(fc_ref, bc_ref, fh_ref, bh_ref, o_ref):
    eps = 1e-10
    rows = lax.broadcasted_iota(jnp.int32, (8, 256), 0)
    valid = rows < 3

    def prep(h):
        s = jnp.sum(h)
        return jnp.clip(h / s, eps, None)

    pfc = prep(fc_ref[...])
    pbc = prep(bc_ref[...])
    pfh = prep(fh_ref[...])
    pbh = prep(bh_ref[...])
    lfh = jnp.log(pfh)
    lbh = jnp.log(pbh)

    def ce(pa, lb):
        return -jnp.sum(jnp.where(valid, pa * lb, 0.0))

    ce_pos = ce(pfc, lfh) + ce(pbc, lbh)
    ce_neg = -(ce(pfc, lbh) + ce(pbc, lfh))
    o_ref[...] = jnp.full((8, 128), 1.0 * ce_pos + 0.5 * ce_neg)


def _otsu_threshold(hist):
    prob = hist / jnp.sum(hist, axis=2, keepdims=True)
    cum_prob = jnp.cumsum(prob, axis=2)
    cum_mean = jnp.cumsum(prob * jnp.arange(256, dtype=jnp.float32)[None, None, :], axis=2)
    global_mean = cum_mean[:, :, -1:]
    numerator = (global_mean * cum_prob - cum_mean) ** 2
    denominator = cum_prob * (1.0 - cum_prob)
    between_class_variance = numerator / denominator
    return jnp.argmax(between_class_variance, axis=2)


def kernel(cam_cln, cam_haz, img_haz):
    mesh = plsc.VectorSubcoreMesh(core_axis_name="c", subcore_axis_name="s")
    cams_flat = jnp.concatenate([cam_cln.reshape(-1), cam_haz.reshape(-1)])
    img_flat = img_haz.reshape(-1)

    k1 = pl.kernel(
        _k1_body,
        out_type=[jax.ShapeDtypeStruct((NW, 256), jnp.float32),
                  jax.ShapeDtypeStruct((NW, 32), jnp.float32)],
        mesh=mesh,
        scratch_types=[pltpu.VMEM((K1_CH,), jnp.float32),
                       pltpu.VMEM((4096,), jnp.float32),
                       pltpu.VMEM((256,), jnp.float32),
                       pltpu.VMEM((32,), jnp.float32)],
        compiler_params=pltpu.CompilerParams(needs_layout_passes=False),
    )
    hist_part, mm = k1(cams_flat, img_flat)

    hist16 = hist_part.reshape(16, 2, 256).sum(axis=1)
    t_cln = _otsu_threshold(hist16[:8].reshape(8, 1, 256))
    t_haz = _otsu_threshold(hist16[8:].reshape(8, 1, 256))
    t_cln_f = t_cln.reshape(8).astype(jnp.float32)
    t_haz_f = t_haz.reshape(8).astype(jnp.float32)

    mn = jnp.min(mm[:, :16])
    mx = jnp.max(mm[:, 16:])
    mn = jnp.where(mn == 0, mn + 0.001, mn)
    width = (mx - mn) / 256

    row_b = jnp.arange(NW) // 4
    params = jnp.stack([
        t_cln_f[row_b], t_haz_f[row_b],
        jnp.broadcast_to(mn, (NW,)), jnp.broadcast_to(mx, (NW,)),
        jnp.broadcast_to(width, (NW,)),
    ], axis=1)
    params = jnp.broadcast_to(params[:, :, None], (NW, 5, 16)).reshape(NW, 80)

    k2 = pl.kernel(
        _k2_body,
        out_type=[jax.ShapeDtypeStruct((NW, 3072), jnp.float32)],
        mesh=mesh,
        scratch_types=[pltpu.VMEM((80,), jnp.float32),
                       pltpu.VMEM((K2_CH,), jnp.float32),
                       pltpu.VMEM((K2_CH,), jnp.float32),
                       pltpu.VMEM((K2_CH,), jnp.float32),
                       pltpu.VMEM((K2_CH,), jnp.float32),
                       pltpu.VMEM((K2_CH,), jnp.float32),
                       pltpu.VMEM((49152,), jnp.float32),
                       pltpu.VMEM((3072,), jnp.float32)],
        compiler_params=pltpu.CompilerParams(needs_layout_passes=False),
    )
    (hist2_part,) = k2(cams_flat, img_flat, params)

    H = hist2_part.sum(axis=0).reshape(3, 4, 256)
    fc = H[:, 2] + H[:, 3]
    bc = H[:, 0] + H[:, 1]
    fh = H[:, 1] + H[:, 3]
    bh = H[:, 0] + H[:, 2]

    def pad8(x):
        return jnp.concatenate([x, jnp.zeros((5, 256), jnp.float32)], axis=0)

    out = pl.pallas_call(
        _ce_body,
        out_shape=jax.ShapeDtypeStruct((8, 128), jnp.float32),
    )(pad8(fc), pad8(bc), pad8(fh), pad8(bh))
    return out[0, 0]


# speedup vs baseline: 90.6771x; 1.6628x over previous
import jax
import jax.numpy as jnp
from jax import lax
from jax.experimental import pallas as pl
from jax.experimental.pallas import tpu as pltpu
from jax.experimental.pallas import tpu_sc as plsc

NW = 32
L = 16
CAM_PIX = 147456
CAM_WORDS = 2 * 8 * CAM_PIX
IMG_WORDS = 3538944
CH = 9216
K1_PER_TILE = CAM_WORDS // NW
K2_PER_TILE = IMG_WORDS // NW


def _tca_body(cc_ref, ch_ref, img_ref, bins_ref, mn_ref, mx_ref):
    width = (255.0 - 0.0) / 256

    def binify(x):
        v = x * 255.0
        return jnp.clip(jnp.floor((v - 0.0) / width), 0, 255).astype(jnp.int32)

    bins_ref[0] = binify(cc_ref[...])
    bins_ref[1] = binify(ch_ref[...])
    img = img_ref[...]
    mn_ref[...] = jnp.min(img).reshape(1, 1)
    mx_ref[...] = jnp.max(img).reshape(1, 1)


def _k1_body(bins_hbm, hist_out, dbuf, hscr, fbuf, s0, s1):
    wid = lax.axis_index("s") * 2 + lax.axis_index("c")
    lane = lax.iota(jnp.int32, L) * 257
    zeros16 = jnp.zeros((L,), jnp.float32)
    ones16 = jnp.ones((L,), jnp.float32)

    def zero_hist(i, _):
        for u in range(8):
            hscr[pl.ds((i * 8 + u) * L, L)] = zeros16
        return 0
    lax.fori_loop(0, 33, zero_hist, 0)

    off = wid * K1_PER_TILE
    sems = (s0, s1)
    nch = K1_PER_TILE // CH

    def start(k):
        return pltpu.async_copy(
            bins_hbm.at[pl.ds(off + k * CH, CH)],
            dbuf.at[pl.ds((k % 2) * CH, CH)], sems[k % 2])

    handles = {0: start(0)}
    for k in range(nch):
        handles[k].wait()
        if k + 1 < nch:
            handles[k + 1] = start(k + 1)
        base = (k % 2) * CH

        def inner(i, _):
            for u in range(8):
                b = dbuf[pl.ds(base + (i * 8 + u) * L, L)]
                plsc.addupdate_scatter(hscr, [lane + b], ones16)
            return 0
        lax.fori_loop(0, CH // (L * 8), inner, 0)

    def red(g, _):
        acc = zeros16
        for l in range(L):
            acc = acc + hscr[pl.ds(l * 257 + g * L, L)]
        fbuf[pl.ds(g * L, L)] = acc
        return 0
    lax.fori_loop(0, 16, red, 0)
    pltpu.sync_copy(fbuf, hist_out.at[wid])


def _tcb_body(tc_ref, th_ref, mn_ref, mx_ref, wd_ref, cc_ref, ch_ref,
              img_ref, pre_ref):
    b = pl.program_id(0)
    tc = tc_ref[b]
    th = th_ref[b]
    mn = mn_ref[0]
    mx = mx_ref[0]
    wd = wd_ref[0]
    cc = cc_ref[0]
    ch = ch_ref[0]
    mcomb = (jnp.where(cc * 255.0 > tc, 512, 0)
             + jnp.where(ch * 255.0 > th, 256, 0)).astype(jnp.int32)
    for c in range(3):
        x = img_ref[0, c]
        valid = (x >= mn) & (x <= mx)
        idx = jnp.clip(jnp.floor((x - mn) / wd), 0, 255).astype(jnp.int32)
        pre_ref[0, c] = jnp.where(valid, idx + mcomb + c * 1024, -1)


def _k2_body(pre_hbm, hist_out, dbuf, hscr, fbuf, s0, s1):
    wid = lax.axis_index("s") * 2 + lax.axis_index("c")
    lane = lax.iota(jnp.int32, L) * 3073
    zeros16 = jnp.zeros((L,), jnp.float32)
    ones16 = jnp.ones((L,), jnp.float32)

    def zero_hist(i, _):
        for u in range(8):
            hscr[pl.ds((i * 8 + u) * L, L)] = zeros16
        return 0
    lax.fori_loop(0, 385, zero_hist, 0)

    off = wid * K2_PER_TILE
    sems = (s0, s1)
    nch = K2_PER_TILE // CH

    def start(k):
        return pltpu.async_copy(
            pre_hbm.at[pl.ds(off + k * CH, CH)],
            dbuf.at[pl.ds((k % 2) * CH, CH)], sems[k % 2])

    handles = {0: start(0)}
    for k in range(nch):
        handles[k].wait()
        if k + 1 < nch:
            handles[k + 1] = start(k + 1)
        base = (k % 2) * CH

        def inner(i, _):
            for u in range(8):
                b = dbuf[pl.ds(base + (i * 8 + u) * L, L)]
                mask = b >= 0
                bm = jnp.maximum(b, 0)
                plsc.addupdate_scatter(hscr, [lane + bm], ones16, mask=mask)
            return 0
        lax.fori_loop(0, CH // (L * 8), inner, 0)

    def red(g, _):
        acc = zeros16
        for l in range(L):
            acc = acc + hscr[pl.ds(l * 3073 + g * L, L)]
        fbuf[pl.ds(g * L, L)] = acc
        return 0
    lax.fori_loop(0, 192, red, 0)
    pltpu.sync_copy(fbuf, hist_out.at[wid])


def _ce_body(fc_ref, bc_ref, fh_ref, bh_ref, o_ref):
    eps = 1e-10
    rows = lax.broadcasted_iota(jnp.int32, (8, 256), 0)
    valid = rows < 3

    def prep(h):
        s = jnp.sum(h)
        return jnp.clip(h / s, eps, None)

    pfc = prep(fc_ref[...])
    pbc = prep(bc_ref[...])
    pfh = prep(fh_ref[...])
    pbh = prep(bh_ref[...])
    lfh = jnp.log(pfh)
    lbh = jnp.log(pbh)

    def ce(pa, lb):
        return -jnp.sum(jnp.where(valid, pa * lb, 0.0))

    ce_pos = ce(pfc, lfh) + ce(pbc, lbh)
    ce_neg = -(ce(pfc, lbh) + ce(pbc, lfh))
    o_ref[...] = jnp.full((8, 128), 1.0 * ce_pos + 0.5 * ce_neg)


def _otsu_threshold(hist):
    prob = hist / jnp.sum(hist, axis=2, keepdims=True)
    cum_prob = jnp.cumsum(prob, axis=2)
    cum_mean = jnp.cumsum(prob * jnp.arange(256, dtype=jnp.float32)[None, None, :], axis=2)
    global_mean = cum_mean[:, :, -1:]
    numerator = (global_mean * cum_prob - cum_mean) ** 2
    denominator = cum_prob * (1.0 - cum_prob)
    between_class_variance = numerator / denominator
    return jnp.argmax(between_class_variance, axis=2)


def kernel(cam_cln, cam_haz, img_haz):
    mesh = plsc.VectorSubcoreMesh(core_axis_name="c", subcore_axis_name="s")
    sc_params = pltpu.CompilerParams(needs_layout_passes=False)

    cc2 = cam_cln.reshape(9216, 128)
    ch2 = cam_haz.reshape(9216, 128)
    img2 = img_haz.reshape(27648, 128)

    cam_bins, mn0, mx0 = pl.pallas_call(
        _tca_body,
        out_shape=[jax.ShapeDtypeStruct((2, 9216, 128), jnp.int32),
                   jax.ShapeDtypeStruct((1, 1), jnp.float32),
                   jax.ShapeDtypeStruct((1, 1), jnp.float32)],
    )(cc2, ch2, img2)

    k1 = pl.kernel(
        _k1_body,
        out_type=[jax.ShapeDtypeStruct((NW, 256), jnp.float32)],
        mesh=mesh,
        scratch_types=[pltpu.VMEM((2 * CH,), jnp.int32),
                       pltpu.VMEM((4224,), jnp.float32),
                       pltpu.VMEM((256,), jnp.float32),
                       pltpu.SemaphoreType.DMA,
                       pltpu.SemaphoreType.DMA],
        compiler_params=sc_params,
    )
    hist_part = k1(cam_bins.reshape(-1))[0]

    hist16 = hist_part.reshape(16, 2, 256).sum(axis=1)
    t_cln = _otsu_threshold(hist16[:8].reshape(8, 1, 256))
    t_haz = _otsu_threshold(hist16[8:].reshape(8, 1, 256))
    t_cln_f = t_cln.reshape(8).astype(jnp.float32)
    t_haz_f = t_haz.reshape(8).astype(jnp.float32)

    mn = mn0[0, 0]
    mx = mx0[0, 0]
    mn = jnp.where(mn == 0, mn + 0.001, mn)
    width = (mx - mn) / 256
    mn1 = mn.reshape(1)
    mx1 = mx.reshape(1)
    wd1 = width.reshape(1)

    cc3 = cam_cln.reshape(8, 1152, 128)
    ch3 = cam_haz.reshape(8, 1152, 128)
    img4 = img_haz.reshape(8, 3, 1152, 128)

    pre = pl.pallas_call(
        _tcb_body,
        grid=(8,),
        in_specs=[
            pl.BlockSpec(memory_space=pltpu.SMEM),
            pl.BlockSpec(memory_space=pltpu.SMEM),
            pl.BlockSpec(memory_space=pltpu.SMEM),
            pl.BlockSpec(memory_space=pltpu.SMEM),
            pl.BlockSpec(memory_space=pltpu.SMEM),
            pl.BlockSpec((1, 1152, 128), lambda b: (b, 0, 0)),
            pl.BlockSpec((1, 1152, 128), lambda b: (b, 0, 0)),
            pl.BlockSpec((1, 3, 1152, 128), lambda b: (b, 0, 0, 0)),
        ],
        out_specs=pl.BlockSpec((1, 3, 1152, 128), lambda b: (b, 0, 0, 0)),
        out_shape=jax.ShapeDtypeStruct((8, 3, 1152, 128), jnp.int32),
    )(t_cln_f, t_haz_f, mn1, mx1, wd1, cc3, ch3, img4)

    k2 = pl.kernel(
        _k2_body,
        out_type=[jax.ShapeDtypeStruct((NW, 3072), jnp.float32)],
        mesh=mesh,
        scratch_types=[pltpu.VMEM((2 * CH,), jnp.int32),
                       pltpu.VMEM((49280,), jnp.float32),
                       pltpu.VMEM((3072,), jnp.float32),
                       pltpu.SemaphoreType.DMA,
                       pltpu.SemaphoreType.DMA],
        compiler_params=sc_params,
    )
    hist2_part = k2(pre.reshape(-1))[0]

    H = hist2_part.sum(axis=0).reshape(3, 4, 256)
    fc = H[:, 2] + H[:, 3]
    bc = H[:, 0] + H[:, 1]
    fh = H[:, 1] + H[:, 3]
    bh = H[:, 0] + H[:, 2]

    def pad8(x):
        return jnp.concatenate([x, jnp.zeros((5, 256), jnp.float32)], axis=0)

    out = pl.pallas_call(
        _ce_body,
        out_shape=jax.ShapeDtypeStruct((8, 128), jnp.float32),
    )(pad8(fc), pad8(bc), pad8(fh), pad8(bh))
    return out[0, 0]


# speedup vs baseline: 94.0936x; 1.0377x over previous
import jax
import jax.numpy as jnp
from jax import lax
from jax.experimental import pallas as pl
from jax.experimental.pallas import tpu as pltpu
from jax.experimental.pallas import tpu_sc as plsc

NW = 32
L = 16
CAM_PIX = 147456
CAM_WORDS = 2 * 8 * CAM_PIX
IMG_WORDS = 3538944
CH = 9216
K1_PER_TILE = CAM_WORDS // NW
K2_PER_TILE = IMG_WORDS // NW


def _tca_body(cc_ref, ch_ref, img_ref, bins_ref, mn_ref, mx_ref):
    g = pl.program_id(0)
    width = (255.0 - 0.0) / 256

    def binify(x):
        v = x * 255.0
        return jnp.clip(jnp.floor((v - 0.0) / width), 0, 255).astype(jnp.int32)

    bins_ref[0] = binify(cc_ref[...])
    bins_ref[1] = binify(ch_ref[...])
    img = img_ref[...]
    bmn = jnp.min(img).reshape(1, 1)
    bmx = jnp.max(img).reshape(1, 1)

    @pl.when(g == 0)
    def _():
        mn_ref[...] = bmn
        mx_ref[...] = bmx

    @pl.when(g > 0)
    def _():
        mn_ref[...] = jnp.minimum(mn_ref[...], bmn)
        mx_ref[...] = jnp.maximum(mx_ref[...], bmx)


def _k1_body(bins_hbm, hist_out, dbuf, hscr, fbuf, s0, s1):
    wid = lax.axis_index("s") * 2 + lax.axis_index("c")
    lane = lax.iota(jnp.int32, L) * 257
    zeros16 = jnp.zeros((L,), jnp.float32)
    ones16 = jnp.ones((L,), jnp.float32)

    def zero_hist(i, _):
        for u in range(8):
            hscr[pl.ds((i * 8 + u) * L, L)] = zeros16
        return 0
    lax.fori_loop(0, 33, zero_hist, 0)

    off = wid * K1_PER_TILE
    sems = (s0, s1)
    nch = K1_PER_TILE // CH

    def start(k):
        return pltpu.async_copy(
            bins_hbm.at[pl.ds(off + k * CH, CH)],
            dbuf.at[pl.ds((k % 2) * CH, CH)], sems[k % 2])

    handles = {0: start(0)}
    for k in range(nch):
        handles[k].wait()
        if k + 1 < nch:
            handles[k + 1] = start(k + 1)
        base = (k % 2) * CH

        def inner(i, _):
            for u in range(8):
                b = dbuf[pl.ds(base + (i * 8 + u) * L, L)]
                plsc.addupdate_scatter(hscr, [lane + b], ones16)
            return 0
        lax.fori_loop(0, CH // (L * 8), inner, 0)

    def red(g, _):
        acc = zeros16
        for l in range(L):
            acc = acc + hscr[pl.ds(l * 257 + g * L, L)]
        fbuf[pl.ds(g * L, L)] = acc
        return 0
    lax.fori_loop(0, 16, red, 0)
    pltpu.sync_copy(fbuf, hist_out.at[wid])


def _tcb_body(tc_ref, th_ref, mn_ref, mx_ref, cc_ref, ch_ref,
              img_ref, pre_ref):
    b = pl.program_id(0)
    tc = tc_ref[b]
    th = th_ref[b]
    mn0 = mn_ref[0]
    mx = mx_ref[0]
    mn = jnp.where(mn0 == 0, mn0 + 0.001, mn0)
    wd = (mx - mn) / 256
    cc = cc_ref[0]
    ch = ch_ref[0]
    mcomb = (jnp.where(cc * 255.0 > tc, 512, 0)
             + jnp.where(ch * 255.0 > th, 256, 0)).astype(jnp.int32)
    for c in range(3):
        x = img_ref[0, c]
        valid = (x >= mn) & (x <= mx)
        idx = jnp.clip(jnp.floor((x - mn) / wd), 0, 255).astype(jnp.int32)
        pre_ref[0, c] = jnp.where(valid, idx + mcomb + c * 1024, 3072)


def _k2_body(pre_hbm, hist_out, dbuf, hscr, fbuf, s0, s1):
    wid = lax.axis_index("s") * 2 + lax.axis_index("c")
    lane = lax.iota(jnp.int32, L) * 3073
    zeros16 = jnp.zeros((L,), jnp.float32)
    ones16 = jnp.ones((L,), jnp.float32)

    def zero_hist(i, _):
        for u in range(8):
            hscr[pl.ds((i * 8 + u) * L, L)] = zeros16
        return 0
    lax.fori_loop(0, 385, zero_hist, 0)

    off = wid * K2_PER_TILE
    sems = (s0, s1)
    nch = K2_PER_TILE // CH

    def start(k):
        return pltpu.async_copy(
            pre_hbm.at[pl.ds(off + k * CH, CH)],
            dbuf.at[pl.ds((k % 2) * CH, CH)], sems[k % 2])

    handles = {0: start(0)}
    for k in range(nch):
        handles[k].wait()
        if k + 1 < nch:
            handles[k + 1] = start(k + 1)
        base = (k % 2) * CH

        def inner(i, _):
            for u in range(8):
                b = dbuf[pl.ds(base + (i * 8 + u) * L, L)]
                plsc.addupdate_scatter(hscr, [lane + b], ones16)
            return 0
        lax.fori_loop(0, CH // (L * 8), inner, 0)

    def red(g, _):
        acc = zeros16
        for l in range(L):
            acc = acc + hscr[pl.ds(l * 3073 + g * L, L)]
        fbuf[pl.ds(g * L, L)] = acc
        return 0
    lax.fori_loop(0, 192, red, 0)
    pltpu.sync_copy(fbuf, hist_out.at[wid])


def _ce_body(h_ref, o_ref):
    eps = 1e-10

    def row(i):
        return h_ref[pl.ds(i, 1), :]

    fc = [row(4 * c + 2) + row(4 * c + 3) for c in range(3)]
    bc = [row(4 * c + 0) + row(4 * c + 1) for c in range(3)]
    fh = [row(4 * c + 1) + row(4 * c + 3) for c in range(3)]
    bh = [row(4 * c + 0) + row(4 * c + 2) for c in range(3)]

    def tot(v):
        return jnp.sum(v[0]) + jnp.sum(v[1]) + jnp.sum(v[2])

    def prep(v):
        s = tot(v)
        return [jnp.clip(x / s, eps, None) for x in v]

    pfc, pbc, pfh, pbh = prep(fc), prep(bc), prep(fh), prep(bh)
    lfh = [jnp.log(x) for x in pfh]
    lbh = [jnp.log(x) for x in pbh]

    def ce(pa, lb):
        return -(jnp.sum(pa[0] * lb[0]) + jnp.sum(pa[1] * lb[1])
                 + jnp.sum(pa[2] * lb[2]))

    ce_pos = ce(pfc, lfh) + ce(pbc, lbh)
    ce_neg = -(ce(pfc, lbh) + ce(pbc, lfh))
    o_ref[...] = jnp.full((8, 128), 1.0 * ce_pos + 0.5 * ce_neg)


def _otsu_threshold(hist):
    prob = hist / jnp.sum(hist, axis=2, keepdims=True)
    cum_prob = jnp.cumsum(prob, axis=2)
    cum_mean = jnp.cumsum(prob * jnp.arange(256, dtype=jnp.float32)[None, None, :], axis=2)
    global_mean = cum_mean[:, :, -1:]
    numerator = (global_mean * cum_prob - cum_mean) ** 2
    denominator = cum_prob * (1.0 - cum_prob)
    between_class_variance = numerator / denominator
    return jnp.argmax(between_class_variance, axis=2)


def kernel(cam_cln, cam_haz, img_haz):
    mesh = plsc.VectorSubcoreMesh(core_axis_name="c", subcore_axis_name="s")
    sc_params = pltpu.CompilerParams(needs_layout_passes=False)

    cc2 = cam_cln.reshape(9216, 128)
    ch2 = cam_haz.reshape(9216, 128)
    img2 = img_haz.reshape(27648, 128)

    cam_bins, mn0, mx0 = pl.pallas_call(
        _tca_body,
        grid=(16,),
        in_specs=[
            pl.BlockSpec((576, 128), lambda g: (g, 0)),
            pl.BlockSpec((576, 128), lambda g: (g, 0)),
            pl.BlockSpec((1728, 128), lambda g: (g, 0)),
        ],
        out_specs=[
            pl.BlockSpec((2, 576, 128), lambda g: (0, g, 0)),
            pl.BlockSpec((1, 1), lambda g: (0, 0)),
            pl.BlockSpec((1, 1), lambda g: (0, 0)),
        ],
        out_shape=[jax.ShapeDtypeStruct((2, 9216, 128), jnp.int32),
                   jax.ShapeDtypeStruct((1, 1), jnp.float32),
                   jax.ShapeDtypeStruct((1, 1), jnp.float32)],
    )(cc2, ch2, img2)

    k1 = pl.kernel(
        _k1_body,
        out_type=[jax.ShapeDtypeStruct((NW, 256), jnp.float32)],
        mesh=mesh,
        scratch_types=[pltpu.VMEM((2 * CH,), jnp.int32),
                       pltpu.VMEM((4224,), jnp.float32),
                       pltpu.VMEM((256,), jnp.float32),
                       pltpu.SemaphoreType.DMA,
                       pltpu.SemaphoreType.DMA],
        compiler_params=sc_params,
    )
    hist_part = k1(cam_bins.reshape(-1))[0]

    hist16 = hist_part.reshape(16, 2, 256).sum(axis=1)
    t_cln = _otsu_threshold(hist16[:8].reshape(8, 1, 256))
    t_haz = _otsu_threshold(hist16[8:].reshape(8, 1, 256))
    t_cln_f = t_cln.reshape(8).astype(jnp.float32)
    t_haz_f = t_haz.reshape(8).astype(jnp.float32)

    mn1 = mn0.reshape(1)
    mx1 = mx0.reshape(1)

    cc3 = cam_cln.reshape(8, 1152, 128)
    ch3 = cam_haz.reshape(8, 1152, 128)
    img4 = img_haz.reshape(8, 3, 1152, 128)

    pre = pl.pallas_call(
        _tcb_body,
        grid=(8,),
        in_specs=[
            pl.BlockSpec(memory_space=pltpu.SMEM),
            pl.BlockSpec(memory_space=pltpu.SMEM),
            pl.BlockSpec(memory_space=pltpu.SMEM),
            pl.BlockSpec(memory_space=pltpu.SMEM),
            pl.BlockSpec((1, 1152, 128), lambda b: (b, 0, 0)),
            pl.BlockSpec((1, 1152, 128), lambda b: (b, 0, 0)),
            pl.BlockSpec((1, 3, 1152, 128), lambda b: (b, 0, 0, 0)),
        ],
        out_specs=pl.BlockSpec((1, 3, 1152, 128), lambda b: (b, 0, 0, 0)),
        out_shape=jax.ShapeDtypeStruct((8, 3, 1152, 128), jnp.int32),
    )(t_cln_f, t_haz_f, mn1, mx1, cc3, ch3, img4)

    k2 = pl.kernel(
        _k2_body,
        out_type=[jax.ShapeDtypeStruct((NW, 3072), jnp.float32)],
        mesh=mesh,
        scratch_types=[pltpu.VMEM((2 * CH,), jnp.int32),
                       pltpu.VMEM((49280,), jnp.float32),
                       pltpu.VMEM((3072,), jnp.float32),
                       pltpu.SemaphoreType.DMA,
                       pltpu.SemaphoreType.DMA],
        compiler_params=sc_params,
    )
    hist2_part = k2(pre.reshape(-1))[0]

    H12 = hist2_part.reshape(32, 12, 256).sum(axis=0)

    out = pl.pallas_call(
        _ce_body,
        out_shape=jax.ShapeDtypeStruct((8, 128), jnp.float32),
    )(H12)
    return out[0, 0]


# speedup vs baseline: 129.6143x; 1.3775x over previous
import jax
import jax.numpy as jnp
from jax import lax
from jax.experimental import pallas as pl
from jax.experimental.pallas import tpu as pltpu
from jax.experimental.pallas import tpu_sc as plsc

NW = 32
L = 16
CAM_PIX = 147456
CAM_WORDS = 2 * 8 * CAM_PIX
IMG_WORDS = 3538944
CH = 9216
K1_PER_TILE = CAM_WORDS // NW
K2_PER_TILE = IMG_WORDS // NW


def _tca_body(cc_ref, ch_ref, img_ref, bins_ref, mn_ref, mx_ref):
    g = pl.program_id(0)
    width = (255.0 - 0.0) / 256

    lane_off = (lax.broadcasted_iota(jnp.int32, (576, 128), 1) % 16) * 257

    def binify(x):
        v = x * 255.0
        b = jnp.clip(jnp.floor((v - 0.0) / width), 0, 255).astype(jnp.int32)
        return b + lane_off

    bins_ref[0] = binify(cc_ref[...])
    bins_ref[1] = binify(ch_ref[...])
    img = img_ref[...]
    bmn = jnp.min(img).reshape(1, 1)
    bmx = jnp.max(img).reshape(1, 1)

    @pl.when(g == 0)
    def _():
        mn_ref[...] = bmn
        mx_ref[...] = bmx

    @pl.when(g > 0)
    def _():
        mn_ref[...] = jnp.minimum(mn_ref[...], bmn)
        mx_ref[...] = jnp.maximum(mx_ref[...], bmx)


def _k1_body(bins_hbm, hist_out, dbuf, hscr, fbuf, s0, s1):
    wid = lax.axis_index("s") * 2 + lax.axis_index("c")
    zeros16 = jnp.zeros((L,), jnp.float32)
    ones16 = jnp.ones((L,), jnp.float32)

    def zero_hist(i, _):
        for u in range(8):
            hscr[pl.ds((i * 8 + u) * L, L)] = zeros16
        return 0
    lax.fori_loop(0, 33, zero_hist, 0)

    off = wid * K1_PER_TILE
    sems = (s0, s1)
    nch = K1_PER_TILE // CH

    def start(k):
        return pltpu.async_copy(
            bins_hbm.at[pl.ds(off + k * CH, CH)],
            dbuf.at[pl.ds((k % 2) * CH, CH)], sems[k % 2])

    handles = {0: start(0)}
    for k in range(nch):
        handles[k].wait()
        if k + 1 < nch:
            handles[k + 1] = start(k + 1)
        base = (k % 2) * CH

        @plsc.parallel_loop(0, CH // (L * 8))
        def inner(i):
            for u in range(8):
                b = dbuf[pl.ds(base + (i * 8 + u) * L, L)]
                plsc.addupdate_scatter(hscr, [b], ones16)

    def red(g, _):
        acc = zeros16
        for l in range(L):
            acc = acc + hscr[pl.ds(l * 257 + g * L, L)]
        fbuf[pl.ds(g * L, L)] = acc
        return 0
    lax.fori_loop(0, 16, red, 0)
    pltpu.sync_copy(fbuf, hist_out.at[wid])


def _tcb_body(tc_ref, th_ref, mn_ref, mx_ref, cc_ref, ch_ref,
              img_ref, pre_ref):
    b = pl.program_id(0)
    tc = tc_ref[b]
    th = th_ref[b]
    mn0 = mn_ref[0]
    mx = mx_ref[0]
    mn = jnp.where(mn0 == 0, mn0 + 0.001, mn0)
    wd = (mx - mn) / 256
    cc = cc_ref[0]
    ch = ch_ref[0]
    mcomb = (jnp.where(cc * 255.0 > tc, 512, 0)
             + jnp.where(ch * 255.0 > th, 256, 0)).astype(jnp.int32)
    lane_off = (lax.broadcasted_iota(jnp.int32, (1152, 128), 1) % 16) * 3073
    for c in range(3):
        x = img_ref[0, c]
        valid = (x >= mn) & (x <= mx)
        idx = jnp.clip(jnp.floor((x - mn) / wd), 0, 255).astype(jnp.int32)
        pre_ref[0, c] = jnp.where(valid, idx + mcomb + c * 1024, 3072) + lane_off


def _k2_body(pre_hbm, hist_out, dbuf, hscr, fbuf, s0, s1):
    wid = lax.axis_index("s") * 2 + lax.axis_index("c")
    zeros16 = jnp.zeros((L,), jnp.float32)
    ones16 = jnp.ones((L,), jnp.float32)

    def zero_hist(i, _):
        for u in range(8):
            hscr[pl.ds((i * 8 + u) * L, L)] = zeros16
        return 0
    lax.fori_loop(0, 385, zero_hist, 0)

    off = wid * K2_PER_TILE
    sems = (s0, s1)
    nch = K2_PER_TILE // CH

    def start(k):
        return pltpu.async_copy(
            pre_hbm.at[pl.ds(off + k * CH, CH)],
            dbuf.at[pl.ds((k % 2) * CH, CH)], sems[k % 2])

    handles = {0: start(0)}
    for k in range(nch):
        handles[k].wait()
        if k + 1 < nch:
            handles[k + 1] = start(k + 1)
        base = (k % 2) * CH

        @plsc.parallel_loop(0, CH // (L * 8))
        def inner(i):
            for u in range(8):
                b = dbuf[pl.ds(base + (i * 8 + u) * L, L)]
                plsc.addupdate_scatter(hscr, [b], ones16)

    def red(g, _):
        acc = zeros16
        for l in range(L):
            acc = acc + hscr[pl.ds(l * 3073 + g * L, L)]
        fbuf[pl.ds(g * L, L)] = acc
        return 0
    lax.fori_loop(0, 192, red, 0)
    pltpu.sync_copy(fbuf, hist_out.at[wid])


def _ce_body(h_ref, o_ref):
    eps = 1e-10

    def row(i):
        return h_ref[pl.ds(i, 1), :]

    fc = [row(4 * c + 2) + row(4 * c + 3) for c in range(3)]
    bc = [row(4 * c + 0) + row(4 * c + 1) for c in range(3)]
    fh = [row(4 * c + 1) + row(4 * c + 3) for c in range(3)]
    bh = [row(4 * c + 0) + row(4 * c + 2) for c in range(3)]

    def tot(v):
        return jnp.sum(v[0]) + jnp.sum(v[1]) + jnp.sum(v[2])

    def prep(v):
        s = tot(v)
        return [jnp.clip(x / s, eps, None) for x in v]

    pfc, pbc, pfh, pbh = prep(fc), prep(bc), prep(fh), prep(bh)
    lfh = [jnp.log(x) for x in pfh]
    lbh = [jnp.log(x) for x in pbh]

    def ce(pa, lb):
        return -(jnp.sum(pa[0] * lb[0]) + jnp.sum(pa[1] * lb[1])
                 + jnp.sum(pa[2] * lb[2]))

    ce_pos = ce(pfc, lfh) + ce(pbc, lbh)
    ce_neg = -(ce(pfc, lbh) + ce(pbc, lfh))
    o_ref[...] = jnp.full((8, 128), 1.0 * ce_pos + 0.5 * ce_neg)


def _otsu_threshold(hist):
    prob = hist / jnp.sum(hist, axis=2, keepdims=True)
    cum_prob = jnp.cumsum(prob, axis=2)
    cum_mean = jnp.cumsum(prob * jnp.arange(256, dtype=jnp.float32)[None, None, :], axis=2)
    global_mean = cum_mean[:, :, -1:]
    numerator = (global_mean * cum_prob - cum_mean) ** 2
    denominator = cum_prob * (1.0 - cum_prob)
    between_class_variance = numerator / denominator
    return jnp.argmax(between_class_variance, axis=2)


def kernel(cam_cln, cam_haz, img_haz):
    mesh = plsc.VectorSubcoreMesh(core_axis_name="c", subcore_axis_name="s")
    sc_params = pltpu.CompilerParams(needs_layout_passes=False)

    cc2 = cam_cln.reshape(9216, 128)
    ch2 = cam_haz.reshape(9216, 128)
    img2 = img_haz.reshape(27648, 128)

    cam_bins, mn0, mx0 = pl.pallas_call(
        _tca_body,
        grid=(16,),
        in_specs=[
            pl.BlockSpec((576, 128), lambda g: (g, 0)),
            pl.BlockSpec((576, 128), lambda g: (g, 0)),
            pl.BlockSpec((1728, 128), lambda g: (g, 0)),
        ],
        out_specs=[
            pl.BlockSpec((2, 576, 128), lambda g: (0, g, 0)),
            pl.BlockSpec((1, 1), lambda g: (0, 0)),
            pl.BlockSpec((1, 1), lambda g: (0, 0)),
        ],
        out_shape=[jax.ShapeDtypeStruct((2, 9216, 128), jnp.int32),
                   jax.ShapeDtypeStruct((1, 1), jnp.float32),
                   jax.ShapeDtypeStruct((1, 1), jnp.float32)],
    )(cc2, ch2, img2)

    k1 = pl.kernel(
        _k1_body,
        out_type=[jax.ShapeDtypeStruct((NW, 256), jnp.float32)],
        mesh=mesh,
        scratch_types=[pltpu.VMEM((2 * CH,), jnp.int32),
                       pltpu.VMEM((4224,), jnp.float32),
                       pltpu.VMEM((256,), jnp.float32),
                       pltpu.SemaphoreType.DMA,
                       pltpu.SemaphoreType.DMA],
        compiler_params=sc_params,
    )
    hist_part = k1(cam_bins.reshape(-1))[0]

    hist16 = hist_part.reshape(16, 2, 256).sum(axis=1)
    t_cln = _otsu_threshold(hist16[:8].reshape(8, 1, 256))
    t_haz = _otsu_threshold(hist16[8:].reshape(8, 1, 256))
    t_cln_f = t_cln.reshape(8).astype(jnp.float32)
    t_haz_f = t_haz.reshape(8).astype(jnp.float32)

    mn1 = mn0.reshape(1)
    mx1 = mx0.reshape(1)

    cc3 = cam_cln.reshape(8, 1152, 128)
    ch3 = cam_haz.reshape(8, 1152, 128)
    img4 = img_haz.reshape(8, 3, 1152, 128)

    pre = pl.pallas_call(
        _tcb_body,
        grid=(8,),
        in_specs=[
            pl.BlockSpec(memory_space=pltpu.SMEM),
            pl.BlockSpec(memory_space=pltpu.SMEM),
            pl.BlockSpec(memory_space=pltpu.SMEM),
            pl.BlockSpec(memory_space=pltpu.SMEM),
            pl.BlockSpec((1, 1152, 128), lambda b: (b, 0, 0)),
            pl.BlockSpec((1, 1152, 128), lambda b: (b, 0, 0)),
            pl.BlockSpec((1, 3, 1152, 128), lambda b: (b, 0, 0, 0)),
        ],
        out_specs=pl.BlockSpec((1, 3, 1152, 128), lambda b: (b, 0, 0, 0)),
        out_shape=jax.ShapeDtypeStruct((8, 3, 1152, 128), jnp.int32),
    )(t_cln_f, t_haz_f, mn1, mx1, cc3, ch3, img4)

    k2 = pl.kernel(
        _k2_body,
        out_type=[jax.ShapeDtypeStruct((NW, 3072), jnp.float32)],
        mesh=mesh,
        scratch_types=[pltpu.VMEM((2 * CH,), jnp.int32),
                       pltpu.VMEM((49280,), jnp.float32),
                       pltpu.VMEM((3072,), jnp.float32),
                       pltpu.SemaphoreType.DMA,
                       pltpu.SemaphoreType.DMA],
        compiler_params=sc_params,
    )
    hist2_part = k2(pre.reshape(-1))[0]

    H12 = hist2_part.reshape(32, 12, 256).sum(axis=0)

    out = pl.pallas_call(
        _ce_body,
        out_shape=jax.ShapeDtypeStruct((8, 128), jnp.float32),
    )(H12)
    return out[0, 0]


# speedup vs baseline: 136.9096x; 1.0563x over previous
import jax
import jax.numpy as jnp
from jax import lax
from jax.experimental import pallas as pl
from jax.experimental.pallas import tpu as pltpu
from jax.experimental.pallas import tpu_sc as plsc

NW = 32
L = 16
CAM_PIX = 147456
CAM_WORDS = 2 * 8 * CAM_PIX
IMG_WORDS = 3538944
CH = 9216
K1_PER_TILE = CAM_WORDS // NW
K2_PER_TILE = IMG_WORDS // NW


def _tca_body(cc_ref, ch_ref, img_ref, bins_ref, mn_ref, mx_ref):
    g = pl.program_id(0)
    width = (255.0 - 0.0) / 256

    lane_off = (lax.broadcasted_iota(jnp.int32, (2304, 128), 1) % 16) * 257

    def binify(x):
        v = x * 255.0
        b = jnp.clip(jnp.floor((v - 0.0) / width), 0, 255).astype(jnp.int32)
        return b + lane_off

    bins_ref[0] = binify(cc_ref[...])
    bins_ref[1] = binify(ch_ref[...])
    img = img_ref[...]
    bmn = jnp.min(img).reshape(1, 1)
    bmx = jnp.max(img).reshape(1, 1)

    @pl.when(g == 0)
    def _():
        mn_ref[...] = bmn
        mx_ref[...] = bmx

    @pl.when(g > 0)
    def _():
        mn_ref[...] = jnp.minimum(mn_ref[...], bmn)
        mx_ref[...] = jnp.maximum(mx_ref[...], bmx)


def _k1_body(bins_hbm, hist_out, dbuf, hscr, fbuf, s0, s1):
    wid = lax.axis_index("s") * 2 + lax.axis_index("c")
    zeros16 = jnp.zeros((L,), jnp.float32)
    ones16 = jnp.ones((L,), jnp.float32)

    def zero_hist(i, _):
        for u in range(8):
            hscr[pl.ds((i * 8 + u) * L, L)] = zeros16
        return 0
    lax.fori_loop(0, 33, zero_hist, 0)

    off = wid * K1_PER_TILE
    sems = (s0, s1)
    nch = K1_PER_TILE // CH

    def start(k):
        return pltpu.async_copy(
            bins_hbm.at[pl.ds(off + k * CH, CH)],
            dbuf.at[pl.ds((k % 2) * CH, CH)], sems[k % 2])

    handles = {0: start(0)}
    for k in range(nch):
        handles[k].wait()
        if k + 1 < nch:
            handles[k + 1] = start(k + 1)
        base = (k % 2) * CH

        @plsc.parallel_loop(0, CH // (L * 8))
        def inner(i):
            for u in range(8):
                b = dbuf[pl.ds(base + (i * 8 + u) * L, L)]
                plsc.addupdate_scatter(hscr, [b], ones16)

    def red(g, _):
        acc = zeros16
        for l in range(L):
            acc = acc + hscr[pl.ds(l * 257 + g * L, L)]
        fbuf[pl.ds(g * L, L)] = acc
        return 0
    lax.fori_loop(0, 16, red, 0)
    pltpu.sync_copy(fbuf, hist_out.at[wid])


def _tcb_body(tc_ref, th_ref, mn_ref, mx_ref, cc_ref, ch_ref,
              img_ref, pre_ref):
    b = pl.program_id(0)
    tc = tc_ref[b]
    th = th_ref[b]
    mn0 = mn_ref[0]
    mx = mx_ref[0]
    mn = jnp.where(mn0 == 0, mn0 + 0.001, mn0)
    wd = (mx - mn) / 256
    cc = cc_ref[0]
    ch = ch_ref[0]
    mcomb = (jnp.where(cc * 255.0 > tc, 512, 0)
             + jnp.where(ch * 255.0 > th, 256, 0)).astype(jnp.int32)
    lane_off = (lax.broadcasted_iota(jnp.int32, (1152, 128), 1) % 16) * 3073
    for c in range(3):
        x = img_ref[0, c]
        valid = (x >= mn) & (x <= mx)
        idx = jnp.clip(jnp.floor((x - mn) / wd), 0, 255).astype(jnp.int32)
        pre_ref[0, c] = jnp.where(valid, idx + mcomb + c * 1024, 3072) + lane_off


def _k2_body(pre_hbm, hist_out, dbuf, hscr, fbuf, s0, s1):
    wid = lax.axis_index("s") * 2 + lax.axis_index("c")
    zeros16 = jnp.zeros((L,), jnp.float32)
    ones16 = jnp.ones((L,), jnp.float32)

    def zero_hist(i, _):
        for u in range(8):
            hscr[pl.ds((i * 8 + u) * L, L)] = zeros16
        return 0
    lax.fori_loop(0, 385, zero_hist, 0)

    off = wid * K2_PER_TILE
    sems = (s0, s1)
    nch = K2_PER_TILE // CH

    def start(k):
        return pltpu.async_copy(
            pre_hbm.at[pl.ds(off + k * CH, CH)],
            dbuf.at[pl.ds((k % 2) * CH, CH)], sems[k % 2])

    handles = {0: start(0)}
    for k in range(nch):
        handles[k].wait()
        if k + 1 < nch:
            handles[k + 1] = start(k + 1)
        base = (k % 2) * CH

        @plsc.parallel_loop(0, CH // (L * 8))
        def inner(i):
            for u in range(8):
                b = dbuf[pl.ds(base + (i * 8 + u) * L, L)]
                plsc.addupdate_scatter(hscr, [b], ones16)

    def red(g, _):
        acc = zeros16
        for l in range(L):
            acc = acc + hscr[pl.ds(l * 3073 + g * L, L)]
        fbuf[pl.ds(g * L, L)] = acc
        return 0
    lax.fori_loop(0, 192, red, 0)
    pltpu.sync_copy(fbuf, hist_out.at[wid])


def _ce_body(hp_ref, o_ref):
    eps = 1e-10
    H = jnp.sum(hp_ref[...].reshape(32, 12, 256), axis=0)

    def row(i):
        return H[i]

    fc = [row(4 * c + 2) + row(4 * c + 3) for c in range(3)]
    bc = [row(4 * c + 0) + row(4 * c + 1) for c in range(3)]
    fh = [row(4 * c + 1) + row(4 * c + 3) for c in range(3)]
    bh = [row(4 * c + 0) + row(4 * c + 2) for c in range(3)]

    def tot(v):
        return jnp.sum(v[0]) + jnp.sum(v[1]) + jnp.sum(v[2])

    def prep(v):
        s = tot(v)
        return [jnp.clip(x / s, eps, None) for x in v]

    pfc, pbc, pfh, pbh = prep(fc), prep(bc), prep(fh), prep(bh)
    lfh = [jnp.log(x) for x in pfh]
    lbh = [jnp.log(x) for x in pbh]

    def ce(pa, lb):
        return -(jnp.sum(pa[0] * lb[0]) + jnp.sum(pa[1] * lb[1])
                 + jnp.sum(pa[2] * lb[2]))

    ce_pos = ce(pfc, lfh) + ce(pbc, lbh)
    ce_neg = -(ce(pfc, lbh) + ce(pbc, lfh))
    o_ref[...] = jnp.full((8, 128), 1.0 * ce_pos + 0.5 * ce_neg)


def _otsu_threshold(hist):
    prob = hist / jnp.sum(hist, axis=2, keepdims=True)
    cum_prob = jnp.cumsum(prob, axis=2)
    cum_mean = jnp.cumsum(prob * jnp.arange(256, dtype=jnp.float32)[None, None, :], axis=2)
    global_mean = cum_mean[:, :, -1:]
    numerator = (global_mean * cum_prob - cum_mean) ** 2
    denominator = cum_prob * (1.0 - cum_prob)
    between_class_variance = numerator / denominator
    return jnp.argmax(between_class_variance, axis=2)


def kernel(cam_cln, cam_haz, img_haz):
    mesh = plsc.VectorSubcoreMesh(core_axis_name="c", subcore_axis_name="s")
    sc_params = pltpu.CompilerParams(needs_layout_passes=False)

    cc2 = cam_cln.reshape(9216, 128)
    ch2 = cam_haz.reshape(9216, 128)
    img2 = img_haz.reshape(27648, 128)

    cam_bins, mn0, mx0 = pl.pallas_call(
        _tca_body,
        grid=(4,),
        in_specs=[
            pl.BlockSpec((2304, 128), lambda g: (g, 0)),
            pl.BlockSpec((2304, 128), lambda g: (g, 0)),
            pl.BlockSpec((6912, 128), lambda g: (g, 0)),
        ],
        out_specs=[
            pl.BlockSpec((2, 2304, 128), lambda g: (0, g, 0)),
            pl.BlockSpec((1, 1), lambda g: (0, 0)),
            pl.BlockSpec((1, 1), lambda g: (0, 0)),
        ],
        out_shape=[jax.ShapeDtypeStruct((2, 9216, 128), jnp.int32),
                   jax.ShapeDtypeStruct((1, 1), jnp.float32),
                   jax.ShapeDtypeStruct((1, 1), jnp.float32)],
    )(cc2, ch2, img2)

    k1 = pl.kernel(
        _k1_body,
        out_type=[jax.ShapeDtypeStruct((NW, 256), jnp.float32)],
        mesh=mesh,
        scratch_types=[pltpu.VMEM((2 * CH,), jnp.int32),
                       pltpu.VMEM((4224,), jnp.float32),
                       pltpu.VMEM((256,), jnp.float32),
                       pltpu.SemaphoreType.DMA,
                       pltpu.SemaphoreType.DMA],
        compiler_params=sc_params,
    )
    hist_part = k1(cam_bins.reshape(-1))[0]

    hist16 = hist_part.reshape(16, 2, 256).sum(axis=1)
    t_all = _otsu_threshold(hist16.reshape(16, 1, 256)).reshape(16).astype(jnp.float32)
    t_cln_f = t_all[:8]
    t_haz_f = t_all[8:]

    mn1 = mn0.reshape(1)
    mx1 = mx0.reshape(1)

    cc3 = cam_cln.reshape(8, 1152, 128)
    ch3 = cam_haz.reshape(8, 1152, 128)
    img4 = img_haz.reshape(8, 3, 1152, 128)

    pre = pl.pallas_call(
        _tcb_body,
        grid=(8,),
        in_specs=[
            pl.BlockSpec(memory_space=pltpu.SMEM),
            pl.BlockSpec(memory_space=pltpu.SMEM),
            pl.BlockSpec(memory_space=pltpu.SMEM),
            pl.BlockSpec(memory_space=pltpu.SMEM),
            pl.BlockSpec((1, 1152, 128), lambda b: (b, 0, 0)),
            pl.BlockSpec((1, 1152, 128), lambda b: (b, 0, 0)),
            pl.BlockSpec((1, 3, 1152, 128), lambda b: (b, 0, 0, 0)),
        ],
        out_specs=pl.BlockSpec((1, 3, 1152, 128), lambda b: (b, 0, 0, 0)),
        out_shape=jax.ShapeDtypeStruct((8, 3, 1152, 128), jnp.int32),
    )(t_cln_f, t_haz_f, mn1, mx1, cc3, ch3, img4)

    k2 = pl.kernel(
        _k2_body,
        out_type=[jax.ShapeDtypeStruct((NW, 3072), jnp.float32)],
        mesh=mesh,
        scratch_types=[pltpu.VMEM((2 * CH,), jnp.int32),
                       pltpu.VMEM((49280,), jnp.float32),
                       pltpu.VMEM((3072,), jnp.float32),
                       pltpu.SemaphoreType.DMA,
                       pltpu.SemaphoreType.DMA],
        compiler_params=sc_params,
    )
    hist2_part = k2(pre.reshape(-1))[0]

    out = pl.pallas_call(
        _ce_body,
        out_shape=jax.ShapeDtypeStruct((8, 128), jnp.float32),
    )(hist2_part)
    return out[0, 0]
